# baseline (speedup 1.0000x reference)
import functools

import jax
import jax.numpy as jnp
from jax.experimental import pallas as pl
from jax.experimental.pallas import tpu as pltpu


def _body(x_ref, wb_ref, b_ref, o_ref, *, H):
    M, WC = x_ref.shape

    row = jax.lax.broadcasted_iota(jnp.int32, (M, WC), 0) % H
    m_prev = (row > 0).astype(jnp.bfloat16)
    m_next = (row < (H - 1)).astype(jnp.bfloat16)

    def conv_bn(a_bf, i):
        p = pltpu.roll(a_bf, 1, axis=0) * m_prev
        n = pltpu.roll(a_bf, M - 1, axis=0) * m_next
        y = jnp.dot(p, wb_ref[3 * i], preferred_element_type=jnp.float32)
        y += jnp.dot(a_bf, wb_ref[3 * i + 1], preferred_element_type=jnp.float32)
        y += jnp.dot(n, wb_ref[3 * i + 2], preferred_element_type=jnp.float32)
        return y + b_ref[i]

    x0 = x_ref[...]
    h1 = jnp.maximum(conv_bn(x0.astype(jnp.bfloat16), 0), 0.0)
    x1 = jnp.maximum(x0 + conv_bn(h1.astype(jnp.bfloat16), 1), 0.0)
    h2 = jnp.maximum(conv_bn(x1.astype(jnp.bfloat16), 2), 0.0)
    x2 = jnp.maximum(x1 + conv_bn(h2.astype(jnp.bfloat16), 3), 0.0)
    o_ref[...] = x2


def kernel(x, b1_w1, b1_scale1, b1_bias1, b1_w2, b1_scale2, b1_bias2,
           b2_w1, b2_scale1, b2_bias1, b2_w2, b2_scale2, b2_bias2):
    N, H, W, C = x.shape
    WC = W * C
    B = 32 if N % 32 == 0 else N
    M = B * H

    ws = jnp.stack([
        b1_w1 * b1_scale1, b1_w2 * b1_scale2,
        b2_w1 * b2_scale1, b2_w2 * b2_scale2,
    ])
    wb6 = jnp.zeros((4, 3, W, C, W, C), jnp.float32)
    for kx in range(3):
        e = jnp.eye(W, k=1 - kx, dtype=jnp.float32)
        wb6 = wb6 + (e[None, None, :, None, :, None]
                     * ws[:, :, kx][:, :, None, :, None, :])
    wb = wb6.astype(jnp.bfloat16).reshape(12, WC, WC)
    bias = jnp.stack([
        jnp.tile(b1_bias1, W), jnp.tile(b1_bias2, W),
        jnp.tile(b2_bias1, W), jnp.tile(b2_bias2, W),
    ]).reshape(4, 1, WC).astype(jnp.float32)

    x2d = x.reshape(N * H, WC)
    out = pl.pallas_call(
        functools.partial(_body, H=H),
        out_shape=jax.ShapeDtypeStruct((N * H, WC), jnp.float32),
        grid=(N // B,),
        in_specs=[
            pl.BlockSpec((M, WC), lambda n: (n, 0)),
            pl.BlockSpec((12, WC, WC), lambda n: (0, 0, 0)),
            pl.BlockSpec((4, 1, WC), lambda n: (0, 0, 0)),
        ],
        out_specs=pl.BlockSpec((M, WC), lambda n: (n, 0)),
        compiler_params=pltpu.CompilerParams(
            dimension_semantics=("parallel",),
            vmem_limit_bytes=48 * 1024 * 1024,
        ),
    )(x2d, wb, bias)
    return out.reshape(N, H, W, C)


# speedup vs baseline: 1.0994x; 1.0994x over previous
import functools

import jax
import jax.numpy as jnp
from jax.experimental import pallas as pl
from jax.experimental.pallas import tpu as pltpu


def _body(x_ref, wb_ref, b_ref, o_ref, *, H):
    M, WC = x_ref.shape

    row = jax.lax.broadcasted_iota(jnp.int32, (M, WC), 0) % H
    m_prev = (row > 0).astype(jnp.bfloat16)
    m_next = (row < (H - 1)).astype(jnp.bfloat16)

    def conv_bn(a_bf, i):
        p = pltpu.roll(a_bf, 1, axis=0) * m_prev
        n = pltpu.roll(a_bf, M - 1, axis=0) * m_next
        y = jnp.dot(p, wb_ref[3 * i], preferred_element_type=jnp.float32)
        y += jnp.dot(a_bf, wb_ref[3 * i + 1], preferred_element_type=jnp.float32)
        y += jnp.dot(n, wb_ref[3 * i + 2], preferred_element_type=jnp.float32)
        return y + b_ref[i]

    x0 = x_ref[...]
    h1 = jnp.maximum(conv_bn(x0.astype(jnp.bfloat16), 0), 0.0)
    x1 = jnp.maximum(x0 + conv_bn(h1.astype(jnp.bfloat16), 1), 0.0)
    h2 = jnp.maximum(conv_bn(x1.astype(jnp.bfloat16), 2), 0.0)
    x2 = jnp.maximum(x1 + conv_bn(h2.astype(jnp.bfloat16), 3), 0.0)
    o_ref[...] = x2


def kernel(x, b1_w1, b1_scale1, b1_bias1, b1_w2, b1_scale2, b1_bias2,
           b2_w1, b2_scale1, b2_bias1, b2_w2, b2_scale2, b2_bias2):
    N, H, W, C = x.shape
    WC = W * C
    B = 32 if N % 32 == 0 else N
    M = B * H

    ws = jnp.stack([
        b1_w1 * b1_scale1, b1_w2 * b1_scale2,
        b2_w1 * b2_scale1, b2_w2 * b2_scale2,
    ]).astype(jnp.bfloat16)
    xi = jax.lax.broadcasted_iota(jnp.int32, (W, W), 0)
    xo = jax.lax.broadcasted_iota(jnp.int32, (W, W), 1)
    kx_of = xi - xo + 1
    wb6 = jnp.zeros((4, 3, W, C, W, C), jnp.bfloat16)
    for kx in range(3):
        m = (kx_of == kx)[None, None, :, None, :, None]
        wb6 = wb6 + jnp.where(m, ws[:, :, kx][:, :, None, :, None, :], 0)
    wb = wb6.reshape(12, WC, WC)
    bias = jnp.stack([
        jnp.tile(b1_bias1, W), jnp.tile(b1_bias2, W),
        jnp.tile(b2_bias1, W), jnp.tile(b2_bias2, W),
    ]).reshape(4, 1, WC).astype(jnp.float32)

    x2d = x.reshape(N * H, WC)
    out = pl.pallas_call(
        functools.partial(_body, H=H),
        out_shape=jax.ShapeDtypeStruct((N * H, WC), jnp.float32),
        grid=(N // B,),
        in_specs=[
            pl.BlockSpec((M, WC), lambda n: (n, 0)),
            pl.BlockSpec((12, WC, WC), lambda n: (0, 0, 0)),
            pl.BlockSpec((4, 1, WC), lambda n: (0, 0, 0)),
        ],
        out_specs=pl.BlockSpec((M, WC), lambda n: (n, 0)),
        compiler_params=pltpu.CompilerParams(
            dimension_semantics=("parallel",),
            vmem_limit_bytes=48 * 1024 * 1024,
        ),
    )(x2d, wb, bias)
    return out.reshape(N, H, W, C)


# speedup vs baseline: 1.8479x; 1.6807x over previous
import functools

import jax
import jax.numpy as jnp
from jax.experimental import pallas as pl
from jax.experimental.pallas import tpu as pltpu


def _body(x_ref, wb_ref, b_ref, o_ref, *, H):
    B, H_, W, C = x_ref.shape
    M, WC = B * H_, W * C

    row = jax.lax.broadcasted_iota(jnp.int32, (M, WC), 0) % H
    m_prev = (row > 0).astype(jnp.bfloat16)
    m_next = (row < (H - 1)).astype(jnp.bfloat16)

    def conv_bn(a_bf, i):
        p = pltpu.roll(a_bf, 1, axis=0) * m_prev
        n = pltpu.roll(a_bf, M - 1, axis=0) * m_next
        y = jnp.dot(p, wb_ref[3 * i], preferred_element_type=jnp.float32)
        y += jnp.dot(a_bf, wb_ref[3 * i + 1], preferred_element_type=jnp.float32)
        y += jnp.dot(n, wb_ref[3 * i + 2], preferred_element_type=jnp.float32)
        return y + b_ref[i]

    x0 = x_ref[...].reshape(M, WC)
    h1 = jnp.maximum(conv_bn(x0.astype(jnp.bfloat16), 0), 0.0)
    x1 = jnp.maximum(x0 + conv_bn(h1.astype(jnp.bfloat16), 1), 0.0)
    h2 = jnp.maximum(conv_bn(x1.astype(jnp.bfloat16), 2), 0.0)
    x2 = jnp.maximum(x1 + conv_bn(h2.astype(jnp.bfloat16), 3), 0.0)
    o_ref[...] = x2.reshape(B, H_, W, C)


def kernel(x, b1_w1, b1_scale1, b1_bias1, b1_w2, b1_scale2, b1_bias2,
           b2_w1, b2_scale1, b2_bias1, b2_w2, b2_scale2, b2_bias2):
    N, H, W, C = x.shape
    WC = W * C
    B = 32 if N % 32 == 0 else N

    eyes = jnp.stack([jnp.eye(W, k=1 - kx, dtype=jnp.float32)
                      for kx in range(3)])
    ws = jnp.stack([
        b1_w1 * b1_scale1, b1_w2 * b1_scale2,
        b2_w1 * b2_scale1, b2_w2 * b2_scale2,
    ])
    wb = jnp.einsum("xab,vyxcd->vyacbd", eyes, ws).astype(
        jnp.bfloat16).reshape(12, WC, WC)
    bias = jnp.stack([
        jnp.tile(b1_bias1, W), jnp.tile(b1_bias2, W),
        jnp.tile(b2_bias1, W), jnp.tile(b2_bias2, W),
    ]).reshape(4, 1, WC).astype(jnp.float32)

    out = pl.pallas_call(
        functools.partial(_body, H=H),
        out_shape=jax.ShapeDtypeStruct((N, H, W, C), jnp.float32),
        grid=(N // B,),
        in_specs=[
            pl.BlockSpec((B, H, W, C), lambda n: (n, 0, 0, 0)),
            pl.BlockSpec((12, WC, WC), lambda n: (0, 0, 0)),
            pl.BlockSpec((4, 1, WC), lambda n: (0, 0, 0)),
        ],
        out_specs=pl.BlockSpec((B, H, W, C), lambda n: (n, 0, 0, 0)),
        compiler_params=pltpu.CompilerParams(
            dimension_semantics=("parallel",),
            vmem_limit_bytes=48 * 1024 * 1024,
        ),
    )(x, wb, bias)
    return out


# speedup vs baseline: 2.2342x; 1.2091x over previous
import functools

import jax
import jax.numpy as jnp
from jax.experimental import pallas as pl
from jax.experimental.pallas import tpu as pltpu


def _body(x_ref, wb_ref, b_ref, o_ref, *, H):
    B, H_, W, C = x_ref.shape
    M, WC = B * H_, W * C

    row = jax.lax.broadcasted_iota(jnp.int32, (M, WC), 0) % H
    m_prev = (row > 0).astype(jnp.bfloat16)
    m_next = (row < (H - 1)).astype(jnp.bfloat16)

    def conv_bn(a_bf, i):
        p = pltpu.roll(a_bf, 1, axis=0) * m_prev
        n = pltpu.roll(a_bf, M - 1, axis=0) * m_next
        y = jnp.dot(p, wb_ref[3 * i], preferred_element_type=jnp.float32)
        y += jnp.dot(a_bf, wb_ref[3 * i + 1], preferred_element_type=jnp.float32)
        y += jnp.dot(n, wb_ref[3 * i + 2], preferred_element_type=jnp.float32)
        return y + b_ref[i]

    x0 = x_ref[...].reshape(M, WC)
    h1 = jnp.maximum(conv_bn(x0.astype(jnp.bfloat16), 0), 0.0)
    x1 = jnp.maximum(x0 + conv_bn(h1.astype(jnp.bfloat16), 1), 0.0)
    h2 = jnp.maximum(conv_bn(x1.astype(jnp.bfloat16), 2), 0.0)
    x2 = jnp.maximum(x1 + conv_bn(h2.astype(jnp.bfloat16), 3), 0.0)
    o_ref[...] = x2.reshape(B, H_, W, C)


def _prep_body(ws_ref, mask_ref, wb_ref, *, W, C):
    WC = W * C
    r = jax.lax.broadcasted_iota(jnp.int32, (WC, C), 0)
    c = jax.lax.broadcasted_iota(jnp.int32, (WC, C), 1)
    p1 = (r % C == c).astype(jnp.bfloat16)
    p2 = p1.T
    acc = jnp.zeros((WC, WC), jnp.float32)
    for kx in range(3):
        m = ws_ref[0, kx].astype(jnp.bfloat16)
        mp = jnp.dot(m, p2, preferred_element_type=jnp.float32)
        t = jnp.dot(p1, mp.astype(jnp.bfloat16),
                    preferred_element_type=jnp.float32)
        acc += t * mask_ref[kx]
    wb_ref[0] = acc.astype(jnp.bfloat16)


def kernel(x, b1_w1, b1_scale1, b1_bias1, b1_w2, b1_scale2, b1_bias2,
           b2_w1, b2_scale1, b2_bias1, b2_w2, b2_scale2, b2_bias2):
    N, H, W, C = x.shape
    WC = W * C
    B = 32 if N % 32 == 0 else N

    ws = jnp.stack([
        b1_w1 * b1_scale1, b1_w2 * b1_scale2,
        b2_w1 * b2_scale1, b2_w2 * b2_scale2,
    ]).reshape(12, 3, C, C)
    xi_blk = jax.lax.broadcasted_iota(jnp.int32, (3, WC, WC), 1) // C
    xo_blk = jax.lax.broadcasted_iota(jnp.int32, (3, WC, WC), 2) // C
    kx_i = jax.lax.broadcasted_iota(jnp.int32, (3, WC, WC), 0)
    masks = (xi_blk - xo_blk + 1 == kx_i).astype(jnp.float32)
    wb = pl.pallas_call(
        functools.partial(_prep_body, W=W, C=C),
        out_shape=jax.ShapeDtypeStruct((12, WC, WC), jnp.bfloat16),
        grid=(12,),
        in_specs=[
            pl.BlockSpec((1, 3, C, C), lambda i: (i, 0, 0, 0)),
            pl.BlockSpec((3, WC, WC), lambda i: (0, 0, 0)),
        ],
        out_specs=pl.BlockSpec((1, WC, WC), lambda i: (i, 0, 0)),
        compiler_params=pltpu.CompilerParams(
            dimension_semantics=("parallel",),
        ),
    )(ws, masks)
    bias = jnp.stack([
        jnp.tile(b1_bias1, W), jnp.tile(b1_bias2, W),
        jnp.tile(b2_bias1, W), jnp.tile(b2_bias2, W),
    ]).reshape(4, 1, WC).astype(jnp.float32)

    out = pl.pallas_call(
        functools.partial(_body, H=H),
        out_shape=jax.ShapeDtypeStruct((N, H, W, C), jnp.float32),
        grid=(N // B,),
        in_specs=[
            pl.BlockSpec((B, H, W, C), lambda n: (n, 0, 0, 0)),
            pl.BlockSpec((12, WC, WC), lambda n: (0, 0, 0)),
            pl.BlockSpec((4, 1, WC), lambda n: (0, 0, 0)),
        ],
        out_specs=pl.BlockSpec((B, H, W, C), lambda n: (n, 0, 0, 0)),
        compiler_params=pltpu.CompilerParams(
            dimension_semantics=("parallel",),
            vmem_limit_bytes=48 * 1024 * 1024,
        ),
    )(x, wb, bias)
    return out


# speedup vs baseline: 4.6991x; 2.1032x over previous
import functools

import jax
import jax.numpy as jnp
from jax.experimental import pallas as pl
from jax.experimental.pallas import tpu as pltpu


def _body(x_ref, wbt_ref, b_ref, o_ref, *, H):
    f32 = jnp.float32

    def conv_bn(a, i):
        outs = []
        for h in range(H):
            y = jnp.dot(wbt_ref[3 * i + 1], a[h], preferred_element_type=f32)
            if h > 0:
                y += jnp.dot(wbt_ref[3 * i], a[h - 1],
                             preferred_element_type=f32)
            if h < H - 1:
                y += jnp.dot(wbt_ref[3 * i + 2], a[h + 1],
                             preferred_element_type=f32)
            outs.append(y + b_ref[i])
        return outs

    bf = jnp.bfloat16
    x0 = [x_ref[h] for h in range(H)]
    h1 = [jnp.maximum(y, 0.0).astype(bf) for y in conv_bn(x0, 0)]
    x1 = [jnp.maximum(x0[h].astype(f32) + y, 0.0)
          for h, y in enumerate(conv_bn(h1, 1))]
    x1b = [v.astype(bf) for v in x1]
    h2 = [jnp.maximum(y, 0.0).astype(bf) for y in conv_bn(x1b, 2)]
    for h, y in enumerate(conv_bn(h2, 3)):
        o_ref[h] = jnp.maximum(x1[h] + y, 0.0)


def _prep_body(ws_ref, mask_ref, wb_ref, *, C):
    WC = mask_ref.shape[1]
    r = jax.lax.broadcasted_iota(jnp.int32, (WC, C), 0)
    c = jax.lax.broadcasted_iota(jnp.int32, (WC, C), 1)
    p1 = (r % C == c).astype(jnp.bfloat16)
    p2 = p1.T
    acc = jnp.zeros((WC, WC), jnp.float32)
    for kx in range(3):
        m = ws_ref[0, kx].astype(jnp.bfloat16)
        mp = jnp.dot(m, p2, preferred_element_type=jnp.float32)
        t = jnp.dot(p1, mp.astype(jnp.bfloat16),
                    preferred_element_type=jnp.float32)
        acc += t * mask_ref[kx]
    wb_ref[0] = acc.astype(jnp.bfloat16)


def kernel(x, b1_w1, b1_scale1, b1_bias1, b1_w2, b1_scale2, b1_bias2,
           b2_w1, b2_scale1, b2_bias1, b2_w2, b2_scale2, b2_bias2):
    N, H, W, C = x.shape
    WC = W * C
    NSPLIT = 2 if N % 256 == 0 else 1
    NL = N // NSPLIT

    ws = jnp.stack([
        b1_w1 * b1_scale1, b1_w2 * b1_scale2,
        b2_w1 * b2_scale1, b2_w2 * b2_scale2,
    ]).reshape(12, 3, C, C).swapaxes(-1, -2)
    xo_blk = jax.lax.broadcasted_iota(jnp.int32, (3, WC, WC), 1) // C
    xi_blk = jax.lax.broadcasted_iota(jnp.int32, (3, WC, WC), 2) // C
    kx_i = jax.lax.broadcasted_iota(jnp.int32, (3, WC, WC), 0)
    masks = (xi_blk - xo_blk + 1 == kx_i).astype(jnp.float32)
    wbt = pl.pallas_call(
        functools.partial(_prep_body, C=C),
        out_shape=jax.ShapeDtypeStruct((12, WC, WC), jnp.bfloat16),
        grid=(12,),
        in_specs=[
            pl.BlockSpec((1, 3, C, C), lambda i: (i, 0, 0, 0)),
            pl.BlockSpec((3, WC, WC), lambda i: (0, 0, 0)),
        ],
        out_specs=pl.BlockSpec((1, WC, WC), lambda i: (i, 0, 0)),
        compiler_params=pltpu.CompilerParams(
            dimension_semantics=("parallel",),
        ),
    )(ws, masks)

    bias = jnp.stack([
        jnp.tile(b1_bias1, W), jnp.tile(b1_bias2, W),
        jnp.tile(b2_bias1, W), jnp.tile(b2_bias2, W),
    ]).reshape(4, WC, 1).astype(jnp.float32)

    xt = x.transpose(1, 2, 3, 0).reshape(H, WC, N).astype(jnp.bfloat16)

    out = pl.pallas_call(
        functools.partial(_body, H=H),
        out_shape=jax.ShapeDtypeStruct((H, WC, N), jnp.float32),
        grid=(NSPLIT,),
        in_specs=[
            pl.BlockSpec((H, WC, NL), lambda j: (0, 0, j)),
            pl.BlockSpec((12, WC, WC), lambda j: (0, 0, 0)),
            pl.BlockSpec((4, WC, 1), lambda j: (0, 0, 0)),
        ],
        out_specs=pl.BlockSpec((H, WC, NL), lambda j: (0, 0, j)),
        compiler_params=pltpu.CompilerParams(
            dimension_semantics=("parallel",),
            vmem_limit_bytes=56 * 1024 * 1024,
        ),
    )(xt, wbt, bias)
    return out.reshape(H, W, C, N).transpose(3, 0, 1, 2)
